# Initial kernel scaffold; baseline (speedup 1.0000x reference)
import jax
import jax.numpy as jnp
from jax.experimental import pallas as pl


def kernel(x, preds):
    raise NotImplementedError("write your pallas kernel here")


# speedup vs baseline: 10.8392x; 10.8392x over previous
import jax
import jax.numpy as jnp
from jax.experimental import pallas as pl

B, C, H, W, K = 8, 192, 128, 128, 19
HW = H * W
NEG_INF = float("-inf")


def _phase_a_body(preds_ref, x_ref, cent_ref, am_ref):
    p = preds_ref[0]
    xv = x_ref[0]
    s = jnp.max(p, axis=0)
    am = jnp.argmax(p, axis=0).astype(jnp.int32)
    kio = jax.lax.broadcasted_iota(jnp.int32, (K, HW), 0)
    mask = am[None, :] == kio
    m = jnp.max(jnp.where(mask, s[None, :], NEG_INF), axis=1)
    m_pp = jnp.sum(jnp.where(mask, m[:, None], 0.0), axis=0)
    e = jnp.exp(s - m_pp)
    d = jnp.sum(jnp.where(mask, e[None, :], 0.0), axis=1)
    d_pp = jnp.sum(jnp.where(mask, d[:, None], 0.0), axis=0)
    wgt = e / d_pp
    mt = jnp.where(mask, wgt[None, :], 0.0).astype(jnp.bfloat16)
    xb = xv.astype(jnp.bfloat16)
    cent = jax.lax.dot_general(
        xb, mt, (((1,), (1,)), ((), ())),
        preferred_element_type=jnp.float32)
    cent_ref[0] = cent
    am_ref[0, 0] = am


def _phase_b_body(cent_ref, am_ref, out_ref):
    cent = cent_ref[0]
    am = am_ref[0, 0]
    idx = jnp.broadcast_to(am[None, :], (cent.shape[0], am.shape[0]))
    out_ref[0] = jnp.take_along_axis(cent, idx, axis=1)


def kernel(x, preds):
    xf = x.reshape(B, C, HW)
    pf = preds.reshape(B, K, HW)

    cent, am = pl.pallas_call(
        _phase_a_body,
        grid=(B,),
        in_specs=[
            pl.BlockSpec((1, K, HW), lambda b: (b, 0, 0)),
            pl.BlockSpec((1, C, HW), lambda b: (b, 0, 0)),
        ],
        out_specs=[
            pl.BlockSpec((1, C, K), lambda b: (b, 0, 0)),
            pl.BlockSpec((1, 1, HW), lambda b: (b, 0, 0)),
        ],
        out_shape=[
            jax.ShapeDtypeStruct((B, C, K), jnp.float32),
            jax.ShapeDtypeStruct((B, 1, HW), jnp.int32),
        ],
    )(pf, xf)

    PB = 4096
    out = pl.pallas_call(
        _phase_b_body,
        grid=(B, HW // PB),
        in_specs=[
            pl.BlockSpec((1, C, K), lambda b, j: (b, 0, 0)),
            pl.BlockSpec((1, 1, PB), lambda b, j: (b, 0, j)),
        ],
        out_specs=pl.BlockSpec((1, C, PB), lambda b, j: (b, 0, j)),
        out_shape=jax.ShapeDtypeStruct((B, C, HW), jnp.float32),
    )(cent, am)

    return out.reshape(B, C, H, W)


# speedup vs baseline: 12.1713x; 1.1229x over previous
import jax
import jax.numpy as jnp
from jax.experimental import pallas as pl

B, C, H, W, K = 8, 192, 128, 128, 19
HW = H * W
NEG_INF = float("-inf")


def _phase_a_body(preds_ref, x_ref, cent_ref, am_ref):
    p = preds_ref[0]
    xv = x_ref[0]
    s = jnp.max(p, axis=0)
    am = jnp.argmax(p, axis=0).astype(jnp.int32)
    kio = jax.lax.broadcasted_iota(jnp.int32, (K, HW), 0)
    mask = am[None, :] == kio
    m = jnp.max(jnp.where(mask, s[None, :], NEG_INF), axis=1)
    m_pp = jnp.sum(jnp.where(mask, m[:, None], 0.0), axis=0)
    e = jnp.exp(s - m_pp)
    d = jnp.sum(jnp.where(mask, e[None, :], 0.0), axis=1)
    d_pp = jnp.sum(jnp.where(mask, d[:, None], 0.0), axis=0)
    wgt = e / d_pp
    mt = jnp.where(mask, wgt[None, :], 0.0).astype(jnp.bfloat16)
    xb = xv.astype(jnp.bfloat16)
    cent = jax.lax.dot_general(
        xb, mt, (((1,), (1,)), ((), ())),
        preferred_element_type=jnp.float32)
    cent_ref[0] = cent
    am_ref[0, 0] = am


def _phase_b_body(cent_ref, am_ref, out_ref):
    cent = cent_ref[0]
    am = am_ref[0, 0]
    kio = jax.lax.broadcasted_iota(jnp.int32, (K, am.shape[0]), 0)
    onehot = (am[None, :] == kio).astype(jnp.bfloat16)
    out_ref[0] = jax.lax.dot_general(
        cent.astype(jnp.bfloat16), onehot, (((1,), (0,)), ((), ())),
        preferred_element_type=jnp.float32)


def kernel(x, preds):
    xf = x.reshape(B, C, HW)
    pf = preds.reshape(B, K, HW)

    cent, am = pl.pallas_call(
        _phase_a_body,
        grid=(B,),
        in_specs=[
            pl.BlockSpec((1, K, HW), lambda b: (b, 0, 0)),
            pl.BlockSpec((1, C, HW), lambda b: (b, 0, 0)),
        ],
        out_specs=[
            pl.BlockSpec((1, C, K), lambda b: (b, 0, 0)),
            pl.BlockSpec((1, 1, HW), lambda b: (b, 0, 0)),
        ],
        out_shape=[
            jax.ShapeDtypeStruct((B, C, K), jnp.float32),
            jax.ShapeDtypeStruct((B, 1, HW), jnp.int32),
        ],
    )(pf, xf)

    PB = 4096
    out = pl.pallas_call(
        _phase_b_body,
        grid=(B, HW // PB),
        in_specs=[
            pl.BlockSpec((1, C, K), lambda b, j: (b, 0, 0)),
            pl.BlockSpec((1, 1, PB), lambda b, j: (b, 0, j)),
        ],
        out_specs=pl.BlockSpec((1, C, PB), lambda b, j: (b, 0, j)),
        out_shape=jax.ShapeDtypeStruct((B, C, HW), jnp.float32),
    )(cent, am)

    return out.reshape(B, C, H, W)


# speedup vs baseline: 33.2241x; 2.7297x over previous
import jax
import jax.numpy as jnp
from jax.experimental import pallas as pl

B, C, H, W, K = 8, 192, 128, 128, 19
HW = H * W
NEG_INF = float("-inf")


def _phase_a_body(preds_ref, x_ref, cent_ref, am_ref):
    p = preds_ref[0].reshape(K, HW)
    xv = x_ref[0].reshape(C, HW)
    s = jnp.max(p, axis=0)
    am = jnp.argmax(p, axis=0).astype(jnp.int32)
    kio = jax.lax.broadcasted_iota(jnp.int32, (K, HW), 0)
    mask = am[None, :] == kio
    m = jnp.max(jnp.where(mask, s[None, :], NEG_INF), axis=1)
    m_pp = jnp.sum(jnp.where(mask, m[:, None], 0.0), axis=0)
    e = jnp.exp(s - m_pp)
    d = jnp.sum(jnp.where(mask, e[None, :], 0.0), axis=1)
    d_pp = jnp.sum(jnp.where(mask, d[:, None], 0.0), axis=0)
    wgt = e / d_pp
    mt = jnp.where(mask, wgt[None, :], 0.0).astype(jnp.bfloat16)
    xb = xv.astype(jnp.bfloat16)
    cent = jax.lax.dot_general(
        xb, mt, (((1,), (1,)), ((), ())),
        preferred_element_type=jnp.float32)
    cent_ref[0] = cent
    am_ref[0, 0] = am


HB = 32
PB = HB * W


def _phase_b_body(cent_ref, am_ref, out_ref):
    cent = cent_ref[0]
    am = am_ref[0, 0]
    kio = jax.lax.broadcasted_iota(jnp.int32, (K, PB), 0)
    onehot = (am[None, :] == kio).astype(jnp.bfloat16)
    res = jax.lax.dot_general(
        cent.astype(jnp.bfloat16), onehot, (((1,), (0,)), ((), ())),
        preferred_element_type=jnp.float32)
    out_ref[0] = res.reshape(C, HB, W)


def kernel(x, preds):
    cent, am = pl.pallas_call(
        _phase_a_body,
        grid=(B,),
        in_specs=[
            pl.BlockSpec((1, K, H, W), lambda b: (b, 0, 0, 0)),
            pl.BlockSpec((1, C, H, W), lambda b: (b, 0, 0, 0)),
        ],
        out_specs=[
            pl.BlockSpec((1, C, K), lambda b: (b, 0, 0)),
            pl.BlockSpec((1, 1, HW), lambda b: (b, 0, 0)),
        ],
        out_shape=[
            jax.ShapeDtypeStruct((B, C, K), jnp.float32),
            jax.ShapeDtypeStruct((B, 1, HW), jnp.int32),
        ],
    )(preds, x)

    out = pl.pallas_call(
        _phase_b_body,
        grid=(B, H // HB),
        in_specs=[
            pl.BlockSpec((1, C, K), lambda b, j: (b, 0, 0)),
            pl.BlockSpec((1, 1, PB), lambda b, j: (b, 0, j)),
        ],
        out_specs=pl.BlockSpec((1, C, HB, W), lambda b, j: (b, 0, j, 0)),
        out_shape=jax.ShapeDtypeStruct((B, C, H, W), jnp.float32),
    )(cent, am)

    return out


# speedup vs baseline: 37.1186x; 1.1172x over previous
import jax
import jax.numpy as jnp
from jax.experimental import pallas as pl

B, C, H, W, K = 8, 192, 128, 128, 19
HW = H * W
NEG_INF = float("-inf")


def _phase_a_body(preds_ref, x_ref, cent_ref, am_ref):
    p = preds_ref[0].reshape(K, HW)
    xb = x_ref[0].astype(jnp.bfloat16).reshape(C, HW)
    s = jnp.max(p, axis=0)
    am = jnp.argmax(p, axis=0).astype(jnp.int32)
    kio = jax.lax.broadcasted_iota(jnp.int32, (K, HW), 0)
    mask = am[None, :] == kio
    m = jnp.max(jnp.where(mask, s[None, :], NEG_INF), axis=1)
    m_pp = jnp.sum(jnp.where(mask, m[:, None], 0.0), axis=0)
    e = jnp.exp(s - m_pp)
    d = jnp.sum(jnp.where(mask, e[None, :], 0.0), axis=1)
    d_pp = jnp.sum(jnp.where(mask, d[:, None], 0.0), axis=0)
    wgt = e / d_pp
    mt = jnp.where(mask, wgt[None, :], 0.0).astype(jnp.bfloat16)
    cent = jax.lax.dot_general(
        xb, mt, (((1,), (1,)), ((), ())),
        preferred_element_type=jnp.float32)
    cent_ref[0] = cent
    am_ref[0, 0] = am


HB = 32
PB = HB * W


def _phase_b_body(cent_ref, am_ref, out_ref):
    cent = cent_ref[0]
    am = am_ref[0, 0]
    kio = jax.lax.broadcasted_iota(jnp.int32, (K, PB), 0)
    onehot = (am[None, :] == kio).astype(jnp.bfloat16)
    res = jax.lax.dot_general(
        cent.astype(jnp.bfloat16), onehot, (((1,), (0,)), ((), ())),
        preferred_element_type=jnp.float32)
    out_ref[0] = res.reshape(C, HB, W)


def kernel(x, preds):
    cent, am = pl.pallas_call(
        _phase_a_body,
        grid=(B,),
        in_specs=[
            pl.BlockSpec((1, K, H, W), lambda b: (b, 0, 0, 0)),
            pl.BlockSpec((1, C, H, W), lambda b: (b, 0, 0, 0)),
        ],
        out_specs=[
            pl.BlockSpec((1, C, K), lambda b: (b, 0, 0)),
            pl.BlockSpec((1, 1, HW), lambda b: (b, 0, 0)),
        ],
        out_shape=[
            jax.ShapeDtypeStruct((B, C, K), jnp.float32),
            jax.ShapeDtypeStruct((B, 1, HW), jnp.int32),
        ],
    )(preds, x)

    out = pl.pallas_call(
        _phase_b_body,
        grid=(B, H // HB),
        in_specs=[
            pl.BlockSpec((1, C, K), lambda b, j: (b, 0, 0)),
            pl.BlockSpec((1, 1, PB), lambda b, j: (b, 0, j)),
        ],
        out_specs=pl.BlockSpec((1, C, HB, W), lambda b, j: (b, 0, j, 0)),
        out_shape=jax.ShapeDtypeStruct((B, C, H, W), jnp.float32),
    )(cent, am)

    return out
